# baseline (speedup 1.0000x reference)
import functools

import jax
import jax.numpy as jnp
from jax import lax
from jax.experimental import pallas as pl
from jax.experimental.pallas import tpu as pltpu
from jax.experimental.pallas import tpu_sc as plsc

N = 10000
E = 160000
DIM = 16
B = 4096
NC = 2
NS = 16
NW = NC * NS
L = 16
CHUNK = 128
PK = 128 // DIM


def _sc_mesh():
    return plsc.VectorSubcoreMesh(core_axis_name="c", subcore_axis_name="s")


def _chunk_of(b_per_w):
    for c in range(min(b_per_w, CHUNK), 0, -1):
        if b_per_w % c == 0:
            return c
    return 1


def _sc_gather(table, idx):
    nrows = idx.shape[0]
    b_per_w = nrows // NW
    chunk = _chunk_of(b_per_w)
    n_chunks = b_per_w // chunk
    idx2 = idx.reshape(NW, n_chunks, chunk)

    @functools.partial(
        pl.kernel,
        mesh=_sc_mesh(),
        out_type=jax.ShapeDtypeStruct((NW, b_per_w, DIM), jnp.float32),
        scratch_types=[
            pltpu.VMEM((n_chunks, chunk), jnp.int32),
            pltpu.VMEM((b_per_w, DIM), jnp.float32),
            pltpu.SemaphoreType.DMA,
        ],
        compiler_params=pltpu.CompilerParams(use_tc_tiling_on_sc=False),
    )
    def k(table_hbm, idx_hbm, out_hbm, idx_v, rows_v, sem):
        wid = lax.axis_index("s") * NC + lax.axis_index("c")
        pltpu.sync_copy(idx_hbm.at[wid], idx_v)

        def fire(j):
            return pltpu.async_copy(
                table_hbm.at[idx_v.at[j]],
                rows_v.at[pl.ds(j * chunk, chunk)],
                sem,
            )

        def body(g, _):
            cps = [fire(g * 8 + b) for b in range(8)]
            for cp in cps:
                cp.wait()
            return 0

        lax.fori_loop(0, n_chunks // 8, body, 0, unroll=False)
        rem = n_chunks % 8
        if rem:
            cps = [fire((n_chunks // 8) * 8 + b) for b in range(rem)]
            for cp in cps:
                cp.wait()
        pltpu.sync_copy(rows_v, out_hbm.at[wid])

    return k(table, idx2)


def _sc_scatter(msg3, dst):
    e_per_w = E // NW
    chunk = _chunk_of(e_per_w)
    n_chunks = e_per_w // chunk
    rows_per_s = N // NS
    dst2 = dst.reshape(NW, n_chunks, chunk)

    @functools.partial(
        pl.kernel,
        mesh=_sc_mesh(),
        out_type=jax.ShapeDtypeStruct((NC, N, DIM), jnp.float32),
        scratch_types=[
            pltpu.VMEM((n_chunks, chunk), jnp.int32),
            pltpu.VMEM((e_per_w, DIM), jnp.float32),
            pltpu.VMEM((rows_per_s, DIM), jnp.float32),
            pltpu.VMEM_SHARED((N, DIM), jnp.float32),
            pltpu.SemaphoreType.DMA,
        ],
        compiler_params=pltpu.CompilerParams(use_tc_tiling_on_sc=False),
    )
    def k(msg_hbm, dst_hbm, out_hbm, dst_v, msg_v, zbuf, acc_sh, sem):
        c = lax.axis_index("c")
        s = lax.axis_index("s")
        wid = s * NC + c

        def zloop(i, _):
            zbuf[i, :] = jnp.zeros((L,), jnp.float32)
            return 0

        lax.fori_loop(0, rows_per_s, zloop, 0)
        pltpu.sync_copy(zbuf, acc_sh.at[pl.ds(s * rows_per_s, rows_per_s)])
        pltpu.sync_copy(dst_hbm.at[wid], dst_v)
        pltpu.sync_copy(msg_hbm.at[wid], msg_v)
        plsc.subcore_barrier()

        def fire(j):
            return pltpu.async_copy(
                msg_v.at[pl.ds(j * chunk, chunk)],
                acc_sh.at[dst_v.at[j]],
                sem,
                add=True,
            )

        def body(g, _):
            cps = [fire(g * 8 + b) for b in range(8)]
            for cp in cps:
                cp.wait()
            return 0

        lax.fori_loop(0, n_chunks // 8, body, 0, unroll=False)
        rem_c = n_chunks % 8
        if rem_c:
            cps = [fire((n_chunks // 8) * 8 + b) for b in range(rem_c)]
            for cp in cps:
                cp.wait()
        plsc.subcore_barrier()
        pltpu.sync_copy(
            acc_sh.at[pl.ds(s * rows_per_s, rows_per_s)],
            out_hbm.at[c, pl.ds(s * rows_per_s, rows_per_s)],
        )

    return k(msg3, dst2)


TE = 1600


def _merge8(parts):
    r, c = parts[0].shape
    return jnp.concatenate([p[:, None, :] for p in parts], axis=1).reshape(r * PK, c)


def _msg_body(ea_ref, xp_ref, w1_ref, b1_ref, w2p_ref, b2r_ref, ri_ref, out_ref):
    ea8 = ea_ref[...]
    xp = xp_ref[...]
    ea = _merge8([ea8[:, 2 * j:2 * j + 2] for j in range(PK)])
    xj = _merge8([xp[:, DIM * j:DIM * (j + 1)] for j in range(PK)])
    h = jnp.maximum(
        jnp.dot(ea, w1_ref[...], preferred_element_type=jnp.float32) + b1_ref[...],
        0.0,
    )
    xj_exp = jnp.dot(xj, ri_ref[...], preferred_element_type=jnp.float32)
    z = xj_exp * jnp.tile(h, (1, L))
    msg = (
        jnp.dot(z, w2p_ref[...], preferred_element_type=jnp.float32)
        + jnp.dot(xj, b2r_ref[...], preferred_element_type=jnp.float32)
    )
    m3 = msg.reshape(TE // PK, PK, DIM)
    out_ref[...] = jnp.concatenate([m3[:, j] for j in range(PK)], axis=1)


def _msg_pallas(ea_d, xp, w1, b1, w2p, b2r, ri):
    grid = E // TE
    return pl.pallas_call(
        _msg_body,
        grid=(grid,),
        in_specs=[
            pl.BlockSpec((TE // PK, DIM), lambda i: (i, 0)),
            pl.BlockSpec((TE // PK, 128), lambda i: (i, 0)),
            pl.BlockSpec((2, DIM), lambda i: (0, 0)),
            pl.BlockSpec((1, DIM), lambda i: (0, 0)),
            pl.BlockSpec((DIM * DIM, DIM), lambda i: (0, 0)),
            pl.BlockSpec((DIM, DIM), lambda i: (0, 0)),
            pl.BlockSpec((DIM, DIM * DIM), lambda i: (0, 0)),
        ],
        out_specs=pl.BlockSpec((TE // PK, 128), lambda i: (i, 0)),
        out_shape=jax.ShapeDtypeStruct((E // PK, 128), jnp.float32),
    )(ea_d, xp, w1, b1, w2p, b2r, ri)


TN = 2000


def _epi_body(p_ref, xq_ref, rootb_ref, biast_ref, out_ref):
    agg = p_ref[0] + p_ref[1]
    xr = jnp.dot(xq_ref[...], rootb_ref[...], preferred_element_type=jnp.float32)
    out_ref[...] = jnp.maximum(agg + xr + biast_ref[...], 0.0)


def _epi_pallas(pp, xq, rootb, biast):
    tnp = N // PK
    return pl.pallas_call(
        _epi_body,
        grid=(1,),
        in_specs=[
            pl.BlockSpec((2, tnp, 128), lambda i: (0, 0, 0)),
            pl.BlockSpec((tnp, 128), lambda i: (0, 0)),
            pl.BlockSpec((128, 128), lambda i: (0, 0)),
            pl.BlockSpec((1, 128), lambda i: (0, 0)),
        ],
        out_specs=pl.BlockSpec((tnp, 128), lambda i: (0, 0)),
        out_shape=jax.ShapeDtypeStruct((N // PK, 128), jnp.float32),
    )(pp, xq, rootb, biast)


def _fc_body(fl_ref, fr_ref, wl_ref, wr_ref, b_ref, out_ref):
    acc = (
        jnp.dot(fl_ref[...], wl_ref[...], preferred_element_type=jnp.float32)
        + jnp.dot(fr_ref[...], wr_ref[...], preferred_element_type=jnp.float32)
        + b_ref[...]
    )
    out_ref[...] = jnp.maximum(acc, 0.0)


def _fc_pallas(fl, fr, wl, wr, b):
    return pl.pallas_call(
        _fc_body,
        grid=(1,),
        in_specs=[
            pl.BlockSpec((B, DIM), lambda i: (0, 0)),
            pl.BlockSpec((B, DIM), lambda i: (0, 0)),
            pl.BlockSpec((DIM, DIM), lambda i: (0, 0)),
            pl.BlockSpec((DIM, DIM), lambda i: (0, 0)),
            pl.BlockSpec((1, DIM), lambda i: (0, 0)),
        ],
        out_specs=pl.BlockSpec((B, DIM), lambda i: (0, 0)),
        out_shape=jax.ShapeDtypeStruct((B, DIM), jnp.float32),
    )(fl, fr, wl, wr, b)


def _prep_conv(w1, b1, w2, b2, root, bias):
    w2p = jnp.transpose(w2.reshape(DIM, DIM, DIM), (1, 0, 2)).reshape(DIM * DIM, DIM)
    b2r = b2.reshape(DIM, DIM)
    rootb = jnp.kron(jnp.eye(PK, dtype=jnp.float32), root)
    biast = jnp.tile(bias.reshape(1, DIM), (1, PK))
    return w1, b1.reshape(1, DIM), w2p, b2r, rootb, biast


def _conv(xq, x_lin, src, dst, ea_d, params, ri):
    w1, b1, w2p, b2r, rootb, biast = params
    xj = _sc_gather(x_lin, src)
    xp = xj.reshape(E // PK, 128)
    msgp = _msg_pallas(ea_d, xp, w1, b1, w2p, b2r, ri)
    parts = _sc_scatter(msgp.reshape(NW, E // NW, DIM), dst)
    pp = parts.reshape(NC, N // PK, 128)
    yq = _epi_pallas(pp, xq, rootb, biast)
    return yq, yq.reshape(N, DIM)


def kernel(x1, edge_index1, edge_attr1, x2, edge_index2, edge_attr2, label,
           c1_W1, c1_b1, c1_W2, c1_b2, c1_root, c1_bias,
           c2_W1, c2_b1, c2_W2, c2_b2, c2_root, c2_bias,
           fc_W, fc_b):
    ri = jnp.kron(jnp.eye(DIM, dtype=jnp.float32), jnp.ones((1, L), jnp.float32))
    p1 = _prep_conv(c1_W1, c1_b1, c1_W2, c1_b2, c1_root, c1_bias)
    p2 = _prep_conv(c2_W1, c2_b1, c2_W2, c2_b2, c2_root, c2_bias)

    def gcn(x, edge_index, edge_attr):
        ea_d = edge_attr.reshape(E // PK, DIM)
        src, dst = edge_index[0], edge_index[1]
        xq = x.reshape(N // PK, 128)
        yq, y_lin = _conv(xq, x, src, dst, ea_d, p1, ri)
        yq2, y_lin2 = _conv(yq, y_lin, src, dst, ea_d, p2, ri)
        return y_lin2

    x_lig = gcn(x1, edge_index1, edge_attr1)
    x_rec = gcn(x2, edge_index2, edge_attr2)
    fl = _sc_gather(x_lig, label[:, 0]).reshape(B, DIM)
    fr = _sc_gather(x_rec, label[:, 1]).reshape(B, DIM)
    return _fc_pallas(fl, fr, fc_W[:DIM], fc_W[DIM:], fc_b.reshape(1, DIM))


# speedup vs baseline: 4.3725x; 4.3725x over previous
import functools

import jax
import jax.numpy as jnp
from jax import lax
from jax.experimental import pallas as pl
from jax.experimental.pallas import tpu as pltpu
from jax.experimental.pallas import tpu_sc as plsc

N = 10000
E = 160000
DIM = 16
B = 4096
NC = 2
NS = 16
NW = NC * NS
L = 16
CHUNK = 128
PK = 128 // DIM


def _sc_mesh():
    return plsc.VectorSubcoreMesh(core_axis_name="c", subcore_axis_name="s")


def _chunk_of(b_per_w):
    for c in range(min(b_per_w, CHUNK), 0, -1):
        if b_per_w % c == 0:
            return c
    return 1


def _sc_gather(table, idx):
    nrows = idx.shape[0]
    b_per_w = nrows // NW
    chunk = _chunk_of(b_per_w)
    n_chunks = b_per_w // chunk
    idx2 = idx.reshape(NW, n_chunks, chunk)

    @functools.partial(
        pl.kernel,
        mesh=_sc_mesh(),
        out_type=jax.ShapeDtypeStruct((NW, b_per_w, DIM), jnp.float32),
        scratch_types=[
            pltpu.VMEM((n_chunks, chunk), jnp.int32),
            pltpu.VMEM((b_per_w, DIM), jnp.float32),
            pltpu.SemaphoreType.DMA,
        ],
        compiler_params=pltpu.CompilerParams(use_tc_tiling_on_sc=False, needs_layout_passes=False),
    )
    def k(table_hbm, idx_hbm, out_hbm, idx_v, rows_v, sem):
        wid = lax.axis_index("s") * NC + lax.axis_index("c")
        pltpu.sync_copy(idx_hbm.at[wid], idx_v)

        def fire(j):
            return pltpu.async_copy(
                table_hbm.at[idx_v.at[j]],
                rows_v.at[pl.ds(j * chunk, chunk)],
                sem,
            )

        def body(g, _):
            cps = [fire(g * 8 + b) for b in range(8)]
            for cp in cps:
                cp.wait()
            return 0

        lax.fori_loop(0, n_chunks // 8, body, 0, unroll=False)
        rem = n_chunks % 8
        if rem:
            cps = [fire((n_chunks // 8) * 8 + b) for b in range(rem)]
            for cp in cps:
                cp.wait()
        pltpu.sync_copy(rows_v, out_hbm.at[wid])

    return k(table, idx2)


BLK = 1000


def _sc_gather_t(table, idx):
    e_per_w = E // NW
    chunk = _chunk_of(BLK)
    cpb = BLK // chunk
    n_chunks = e_per_w // chunk
    n_blk = e_per_w // BLK
    ngrp = BLK // L
    rem = BLK - ngrp * L
    idx2 = idx.reshape(NW, n_chunks, chunk)

    @functools.partial(
        pl.kernel,
        mesh=_sc_mesh(),
        out_type=jax.ShapeDtypeStruct((DIM, E), jnp.float32),
        scratch_types=[
            pltpu.VMEM((n_chunks, chunk), jnp.int32),
            pltpu.VMEM((BLK, DIM), jnp.float32),
            pltpu.VMEM((DIM, BLK + L), jnp.float32),
            pltpu.SemaphoreType.DMA,
            pltpu.SemaphoreType.DMA,
        ],
        compiler_params=pltpu.CompilerParams(use_tc_tiling_on_sc=False, needs_layout_passes=False),
    )
    def k(table_hbm, idx_hbm, out_hbm, idx_v, rows_v, xjt_v, sem, sem2):
        wid = lax.axis_index("s") * NC + lax.axis_index("c")
        pltpu.sync_copy(idx_hbm.at[wid], idx_v)
        iota = jax.lax.broadcasted_iota(jnp.int32, (L,), 0)

        def do_block(blk, _):
            cps = [
                pltpu.async_copy(
                    table_hbm.at[idx_v.at[blk * cpb + c]],
                    rows_v.at[pl.ds(c * chunk, chunk)],
                    sem,
                )
                for c in range(cpb)
            ]
            for cp in cps:
                cp.wait()

            def grp(g, _):
                ridx = g * L + iota
                for i in range(DIM):
                    v = plsc.load_gather(rows_v, [ridx, jnp.full((L,), i, jnp.int32)])
                    xjt_v[i, pl.ds(g * L, L)] = v
                return 0

            lax.fori_loop(0, ngrp, grp, 0)
            if rem:
                mask = iota < rem
                ridx = jnp.minimum(ngrp * L + iota, BLK - 1)
                for i in range(DIM):
                    v = plsc.load_gather(
                        rows_v, [ridx, jnp.full((L,), i, jnp.int32)], mask=mask
                    )
                    old = xjt_v[i, pl.ds(ngrp * L, L)]
                    xjt_v[i, pl.ds(ngrp * L, L)] = jnp.where(mask, v, old)
            cps2 = [
                pltpu.async_copy(
                    xjt_v.at[i, pl.ds(0, BLK)],
                    out_hbm.at[i, pl.ds(wid * e_per_w + blk * BLK, BLK)],
                    sem2,
                )
                for i in range(DIM)
            ]
            for cp in cps2:
                cp.wait()
            return 0

        lax.fori_loop(0, n_blk, do_block, 0)

    return k(table, idx2)


def _sc_scatter(msgT, dst):
    e_per_w = E // NW
    chunk = _chunk_of(BLK)
    cpb = BLK // chunk
    n_chunks = e_per_w // chunk
    n_blk = e_per_w // BLK
    ngrp = BLK // L
    rem = BLK - ngrp * L
    rows_per_s = N // NS
    dst2 = dst.reshape(NW, n_chunks, chunk)

    @functools.partial(
        pl.kernel,
        mesh=_sc_mesh(),
        out_type=jax.ShapeDtypeStruct((NC, N, DIM), jnp.float32),
        scratch_types=[
            pltpu.VMEM((n_chunks, chunk), jnp.int32),
            pltpu.VMEM((DIM, BLK), jnp.float32),
            pltpu.VMEM((BLK, DIM), jnp.float32),
            pltpu.VMEM((rows_per_s, DIM), jnp.float32),
            pltpu.VMEM_SHARED((N, DIM), jnp.float32),
            pltpu.SemaphoreType.DMA,
            pltpu.SemaphoreType.DMA,
        ],
        compiler_params=pltpu.CompilerParams(use_tc_tiling_on_sc=False, needs_layout_passes=False),
    )
    def k(msg_hbm, dst_hbm, out_hbm, dst_v, msgt_v, rows_v, zbuf, acc_sh, sem, sem2):
        c = lax.axis_index("c")
        s = lax.axis_index("s")
        wid = s * NC + c
        iota = jax.lax.broadcasted_iota(jnp.int32, (L,), 0)

        def zloop(i, _):
            zbuf[i, :] = jnp.zeros((L,), jnp.float32)
            return 0

        lax.fori_loop(0, rows_per_s, zloop, 0)
        pltpu.sync_copy(zbuf, acc_sh.at[pl.ds(s * rows_per_s, rows_per_s)])
        pltpu.sync_copy(dst_hbm.at[wid], dst_v)
        plsc.subcore_barrier()

        def do_block(blk, _):
            cps = [
                pltpu.async_copy(
                    msg_hbm.at[i, pl.ds(wid * e_per_w + blk * BLK, BLK)],
                    msgt_v.at[i],
                    sem,
                )
                for i in range(DIM)
            ]
            for cp in cps:
                cp.wait()

            def grp(g, _):
                cidx = g * L + iota
                for i in range(DIM):
                    v = plsc.load_gather(
                        msgt_v, [jnp.full((L,), i, jnp.int32), cidx]
                    )
                    plsc.store_scatter(
                        rows_v, [cidx, jnp.full((L,), i, jnp.int32)], v
                    )
                return 0

            lax.fori_loop(0, ngrp, grp, 0)
            if rem:
                mask = iota < rem
                cidx = jnp.minimum(ngrp * L + iota, BLK - 1)
                for i in range(DIM):
                    v = plsc.load_gather(
                        msgt_v, [jnp.full((L,), i, jnp.int32), cidx], mask=mask
                    )
                    plsc.store_scatter(
                        rows_v, [cidx, jnp.full((L,), i, jnp.int32)], v, mask=mask
                    )
            cps2 = [
                pltpu.async_copy(
                    rows_v.at[pl.ds(cc * chunk, chunk)],
                    acc_sh.at[dst_v.at[blk * cpb + cc]],
                    sem2,
                    add=True,
                )
                for cc in range(cpb)
            ]
            for cp in cps2:
                cp.wait()
            return 0

        lax.fori_loop(0, n_blk, do_block, 0)
        plsc.subcore_barrier()
        pltpu.sync_copy(
            acc_sh.at[pl.ds(s * rows_per_s, rows_per_s)],
            out_hbm.at[c, pl.ds(s * rows_per_s, rows_per_s)],
        )

    return k(msgT, dst2)


TE = 3200


def _msg_body(eat_ref, xjt_ref, w1t_ref, b1c_ref, w2pt_ref, b2rt_ref, rit_ref, out_ref):
    eat = eat_ref[...]
    xjt = xjt_ref[...]
    ht = jnp.maximum(
        jnp.dot(w1t_ref[...], eat, preferred_element_type=jnp.float32)
        + b1c_ref[...],
        0.0,
    )
    xj_expt = jnp.dot(rit_ref[...], xjt, preferred_element_type=jnp.float32)
    zt = xj_expt * jnp.tile(ht, (DIM, 1))
    out_ref[...] = (
        jnp.dot(w2pt_ref[...], zt, preferred_element_type=jnp.float32)
        + jnp.dot(b2rt_ref[...], xjt, preferred_element_type=jnp.float32)
    )


def _msg_pallas(eat, xjt, w1t, b1c, w2pt, b2rt, rit):
    grid = E // TE
    return pl.pallas_call(
        _msg_body,
        grid=(grid,),
        in_specs=[
            pl.BlockSpec((2, TE), lambda i: (0, i)),
            pl.BlockSpec((DIM, TE), lambda i: (0, i)),
            pl.BlockSpec((DIM, 2), lambda i: (0, 0)),
            pl.BlockSpec((DIM, 1), lambda i: (0, 0)),
            pl.BlockSpec((DIM, DIM * DIM), lambda i: (0, 0)),
            pl.BlockSpec((DIM, DIM), lambda i: (0, 0)),
            pl.BlockSpec((DIM * DIM, DIM), lambda i: (0, 0)),
        ],
        out_specs=pl.BlockSpec((DIM, TE), lambda i: (0, i)),
        out_shape=jax.ShapeDtypeStruct((DIM, E), jnp.float32),
    )(eat, xjt, w1t, b1c, w2pt, b2rt, rit)


TN = 2000


def _epi_body(p_ref, xq_ref, rootb_ref, biast_ref, out_ref):
    agg = p_ref[0] + p_ref[1]
    xr = jnp.dot(xq_ref[...], rootb_ref[...], preferred_element_type=jnp.float32)
    out_ref[...] = jnp.maximum(agg + xr + biast_ref[...], 0.0)


def _epi_pallas(pp, xq, rootb, biast):
    tnp = N // PK
    return pl.pallas_call(
        _epi_body,
        grid=(1,),
        in_specs=[
            pl.BlockSpec((2, tnp, 128), lambda i: (0, 0, 0)),
            pl.BlockSpec((tnp, 128), lambda i: (0, 0)),
            pl.BlockSpec((128, 128), lambda i: (0, 0)),
            pl.BlockSpec((1, 128), lambda i: (0, 0)),
        ],
        out_specs=pl.BlockSpec((tnp, 128), lambda i: (0, 0)),
        out_shape=jax.ShapeDtypeStruct((N // PK, 128), jnp.float32),
    )(pp, xq, rootb, biast)


def _fc_body(fl_ref, fr_ref, wl_ref, wr_ref, b_ref, out_ref):
    acc = (
        jnp.dot(fl_ref[...], wl_ref[...], preferred_element_type=jnp.float32)
        + jnp.dot(fr_ref[...], wr_ref[...], preferred_element_type=jnp.float32)
        + b_ref[...]
    )
    out_ref[...] = jnp.maximum(acc, 0.0)


def _fc_pallas(fl, fr, wl, wr, b):
    return pl.pallas_call(
        _fc_body,
        grid=(1,),
        in_specs=[
            pl.BlockSpec((B, DIM), lambda i: (0, 0)),
            pl.BlockSpec((B, DIM), lambda i: (0, 0)),
            pl.BlockSpec((DIM, DIM), lambda i: (0, 0)),
            pl.BlockSpec((DIM, DIM), lambda i: (0, 0)),
            pl.BlockSpec((1, DIM), lambda i: (0, 0)),
        ],
        out_specs=pl.BlockSpec((B, DIM), lambda i: (0, 0)),
        out_shape=jax.ShapeDtypeStruct((B, DIM), jnp.float32),
    )(fl, fr, wl, wr, b)


def _prep_conv(w1, b1, w2, b2, root, bias):
    w2p = jnp.transpose(w2.reshape(DIM, DIM, DIM), (1, 0, 2)).reshape(DIM * DIM, DIM)
    rootb = jnp.kron(jnp.eye(PK, dtype=jnp.float32), root)
    biast = jnp.tile(bias.reshape(1, DIM), (1, PK))
    return w1.T, b1.reshape(DIM, 1), w2p.T, b2.reshape(DIM, DIM).T, rootb, biast


def _conv(xq, x_lin, src, dst, eat, params, rit):
    w1t, b1c, w2pt, b2rt, rootb, biast = params
    xjt = _sc_gather_t(x_lin, src)
    msgt = _msg_pallas(eat, xjt, w1t, b1c, w2pt, b2rt, rit)
    parts = _sc_scatter(msgt, dst)
    pp = parts.reshape(NC, N // PK, 128)
    yq = _epi_pallas(pp, xq, rootb, biast)
    return yq, yq.reshape(N, DIM)


def kernel(x1, edge_index1, edge_attr1, x2, edge_index2, edge_attr2, label,
           c1_W1, c1_b1, c1_W2, c1_b2, c1_root, c1_bias,
           c2_W1, c2_b1, c2_W2, c2_b2, c2_root, c2_bias,
           fc_W, fc_b):
    ri = jnp.kron(jnp.eye(DIM, dtype=jnp.float32), jnp.ones((1, L), jnp.float32))
    rit = ri.T
    p1 = _prep_conv(c1_W1, c1_b1, c1_W2, c1_b2, c1_root, c1_bias)
    p2 = _prep_conv(c2_W1, c2_b1, c2_W2, c2_b2, c2_root, c2_bias)

    def gcn(x, edge_index, edge_attr):
        eat = edge_attr.reshape(E, 2).T
        src, dst = edge_index[0], edge_index[1]
        xq = x.reshape(N // PK, 128)
        yq, y_lin = _conv(xq, x, src, dst, eat, p1, rit)
        yq2, y_lin2 = _conv(yq, y_lin, src, dst, eat, p2, rit)
        return y_lin2

    x_lig = gcn(x1, edge_index1, edge_attr1)
    x_rec = gcn(x2, edge_index2, edge_attr2)
    fl = _sc_gather(x_lig, label[:, 0]).reshape(B, DIM)
    fr = _sc_gather(x_rec, label[:, 1]).reshape(B, DIM)
    return _fc_pallas(fl, fr, fc_W[:DIM], fc_W[DIM:], fc_b.reshape(1, DIM))
